# baseline (speedup 1.0000x reference)
import functools

import jax
import jax.numpy as jnp
from jax.experimental import pallas as pl
from jax.experimental.pallas import tpu as pltpu

L = 8
S = 16
CB = 8


def _roll_kernel(lab_ref, x_ref, o_ref, buf, isems, osems, *, kblocks, total, h, w):
    t = pl.program_id(0)

    def in_copy(tt):
        return pltpu.make_async_copy(
            x_ref.at[tt // kblocks, pl.ds((tt % kblocks) * CB, CB)],
            buf.at[tt % S],
            isems.at[tt % S],
        )

    def out_copy(tt):
        return pltpu.make_async_copy(
            buf.at[tt % S],
            o_ref.at[tt // kblocks, pl.ds((tt % kblocks) * CB, CB)],
            osems.at[tt % S],
        )

    @pl.when(t == 0)
    def _():
        for j in range(L):
            in_copy(j).start()

    @pl.when(t + L < total)
    def _():
        @pl.when(t + L >= S)
        def _():
            out_copy(t + L - S).wait()

        in_copy(t + L).start()

    in_copy(t).wait()

    slot = t % S
    c_base = (t % kblocks) * CB
    for i in range(CB):
        lab = lab_ref[c_base + i]
        x = buf[slot, i]

        def roll_h_p(x=x, i=i):
            buf[slot, i] = pltpu.roll(x, 1, 0)

        def roll_h_m(x=x, i=i):
            buf[slot, i] = pltpu.roll(x, h - 1, 0)

        def roll_w_p(x=x, i=i):
            buf[slot, i] = pltpu.roll(x, 1, 1)

        def roll_w_m(x=x, i=i):
            buf[slot, i] = pltpu.roll(x, w - 1, 1)

        jax.lax.switch(lab, [roll_h_p, roll_h_m, roll_w_p, roll_w_m])

    out_copy(t).start()

    @pl.when(t == total - 1)
    def _():
        for j in range(S):
            out_copy(total - S + j).wait()


@jax.jit
def kernel(x, indices):
    b, c, h, w = x.shape
    q = c // 4
    kblocks = c // CB
    total = b * kblocks
    idx = indices.astype(jnp.int32)
    labels = jnp.zeros((c,), jnp.int32).at[idx].set(jnp.arange(c, dtype=jnp.int32) // q)
    grid_spec = pltpu.PrefetchScalarGridSpec(
        num_scalar_prefetch=1,
        grid=(total,),
        in_specs=[pl.BlockSpec(memory_space=pl.ANY)],
        out_specs=pl.BlockSpec(memory_space=pl.ANY),
        scratch_shapes=[
            pltpu.VMEM((S, CB, h, w), jnp.float32),
            pltpu.SemaphoreType.DMA((S,)),
            pltpu.SemaphoreType.DMA((S,)),
        ],
    )
    return pl.pallas_call(
        functools.partial(_roll_kernel, kblocks=kblocks, total=total, h=h, w=w),
        grid_spec=grid_spec,
        out_shape=jax.ShapeDtypeStruct((b, c, h, w), x.dtype),
    )(labels, x)


# speedup vs baseline: 1.0034x; 1.0034x over previous
import functools

import jax
import jax.numpy as jnp
from jax.experimental import pallas as pl
from jax.experimental.pallas import tpu as pltpu

L = 4
S = 8
CB = 16


def _roll_kernel(lab_ref, x_ref, o_ref, buf, isems, osems, *, kblocks, total, h, w):
    t = pl.program_id(0)

    def in_copy(tt):
        return pltpu.make_async_copy(
            x_ref.at[tt // kblocks, pl.ds((tt % kblocks) * CB, CB)],
            buf.at[tt % S],
            isems.at[tt % S],
        )

    def out_copy(tt):
        return pltpu.make_async_copy(
            buf.at[tt % S],
            o_ref.at[tt // kblocks, pl.ds((tt % kblocks) * CB, CB)],
            osems.at[tt % S],
        )

    @pl.when(t == 0)
    def _():
        for j in range(L):
            in_copy(j).start()

    @pl.when(t + L < total)
    def _():
        @pl.when(t + L >= S)
        def _():
            out_copy(t + L - S).wait()

        in_copy(t + L).start()

    in_copy(t).wait()

    slot = t % S
    c_base = (t % kblocks) * CB
    for i in range(CB):
        lab = lab_ref[c_base + i]
        x = buf[slot, i]

        def roll_h_p(x=x, i=i):
            buf[slot, i] = pltpu.roll(x, 1, 0)

        def roll_h_m(x=x, i=i):
            buf[slot, i] = pltpu.roll(x, h - 1, 0)

        def roll_w_p(x=x, i=i):
            buf[slot, i] = pltpu.roll(x, 1, 1)

        def roll_w_m(x=x, i=i):
            buf[slot, i] = pltpu.roll(x, w - 1, 1)

        jax.lax.switch(lab, [roll_h_p, roll_h_m, roll_w_p, roll_w_m])

    out_copy(t).start()

    @pl.when(t == total - 1)
    def _():
        for j in range(S):
            out_copy(total - S + j).wait()


@jax.jit
def kernel(x, indices):
    b, c, h, w = x.shape
    q = c // 4
    kblocks = c // CB
    total = b * kblocks
    idx = indices.astype(jnp.int32)
    labels = jnp.zeros((c,), jnp.int32).at[idx].set(jnp.arange(c, dtype=jnp.int32) // q)
    grid_spec = pltpu.PrefetchScalarGridSpec(
        num_scalar_prefetch=1,
        grid=(total,),
        in_specs=[pl.BlockSpec(memory_space=pl.ANY)],
        out_specs=pl.BlockSpec(memory_space=pl.ANY),
        scratch_shapes=[
            pltpu.VMEM((S, CB, h, w), jnp.float32),
            pltpu.SemaphoreType.DMA((S,)),
            pltpu.SemaphoreType.DMA((S,)),
        ],
    )
    return pl.pallas_call(
        functools.partial(_roll_kernel, kblocks=kblocks, total=total, h=h, w=w),
        grid_spec=grid_spec,
        out_shape=jax.ShapeDtypeStruct((b, c, h, w), x.dtype),
    )(labels, x)
